# Initial kernel scaffold; baseline (speedup 1.0000x reference)
import jax
import jax.numpy as jnp
from jax.experimental import pallas as pl


def kernel(user_emb, item_emb, edge_weight, edge_index, users, pos, neg):
    raise NotImplementedError("write your pallas kernel here")


# speedup vs baseline: 8.2902x; 8.2902x over previous
import functools

import jax
import jax.numpy as jnp
from jax import lax
from jax.experimental import pallas as pl
from jax.experimental.pallas import tpu as pltpu
from jax.experimental.pallas import tpu_sc as plsc

N_USERS = 25000
N_NODES = 50000
N_EDGES = 1600000
D = 32
BATCH = 4096

NC = 2
NS = 16
NW = NC * NS

N_PAD = 50176
ROWS_PER_TILE = N_PAD // NS
E_PAD = 1638400
E_ROWS = E_PAD // 128
ROWS_PER_WORKER = E_ROWS // NW
CHUNK_ROWS = 4
N_CHUNKS = ROWS_PER_WORKER // CHUNK_ROWS

_mesh = plsc.VectorSubcoreMesh(core_axis_name="c", subcore_axis_name="s")


def _spmm_body(t_hbm, src_hbm, dst_hbm, w_hbm, out0, out1,
               acc, zbuf, rb, sb, db, wb, gsem, ssem):
    c = lax.axis_index("c")
    s = lax.axis_index("s")
    wid = c * NS + s

    zero16 = jnp.zeros((16,), jnp.float32)

    def zfill(i, _):
        zbuf[i, pl.ds(0, 16)] = zero16
        zbuf[i, pl.ds(16, 16)] = zero16
        return 0

    lax.fori_loop(0, zbuf.shape[0], zfill, 0)
    r0 = s * ROWS_PER_TILE
    for r in range(ROWS_PER_TILE // zbuf.shape[0]):
        pltpu.sync_copy(zbuf, acc.at[pl.ds(r0 + r * zbuf.shape[0], zbuf.shape[0])])
    plsc.subcore_barrier()

    base_row = wid * ROWS_PER_WORKER

    def chunk_body(g, _):
        row0 = base_row + g * CHUNK_ROWS
        pltpu.sync_copy(src_hbm.at[pl.ds(row0, CHUNK_ROWS)], sb)
        pltpu.sync_copy(dst_hbm.at[pl.ds(row0, CHUNK_ROWS)], db)
        pltpu.sync_copy(w_hbm.at[pl.ds(row0, CHUNK_ROWS)], wb)
        descs = [
            pltpu.async_copy(t_hbm.at[sb.at[j]], rb.at[pl.ds(j * 128, 128)], gsem)
            for j in range(CHUNK_ROWS)
        ]
        for d in descs:
            d.wait()

        def mul_body(t, _):
            j = t >> 3
            i0 = (t & 7) * 16
            wvec = wb[j, pl.ds(i0, 16)]
            e0 = t * 16
            for k in range(16):
                wv = jnp.full((16,), wvec[k], jnp.float32)
                rb[e0 + k, pl.ds(0, 16)] = rb[e0 + k, pl.ds(0, 16)] * wv
                rb[e0 + k, pl.ds(16, 16)] = rb[e0 + k, pl.ds(16, 16)] * wv
            return 0

        lax.fori_loop(0, CHUNK_ROWS * 8, mul_body, 0)

        sdescs = [
            pltpu.async_copy(rb.at[pl.ds(j * 128, 128)], acc.at[db.at[j]], ssem,
                             add=True)
            for j in range(CHUNK_ROWS)
        ]
        for d in sdescs:
            d.wait()
        return 0

    lax.fori_loop(0, N_CHUNKS, chunk_body, 0)
    plsc.subcore_barrier()

    @pl.when(c == 0)
    def _():
        pltpu.sync_copy(acc.at[pl.ds(r0, ROWS_PER_TILE)],
                        out0.at[pl.ds(r0, ROWS_PER_TILE)])

    @pl.when(c == 1)
    def _():
        pltpu.sync_copy(acc.at[pl.ds(r0, ROWS_PER_TILE)],
                        out1.at[pl.ds(r0, ROWS_PER_TILE)])


_sc_params = pltpu.CompilerParams(use_tc_tiling_on_sc=False)

_spmm = pl.kernel(
    _spmm_body,
    out_type=(jax.ShapeDtypeStruct((N_PAD, D), jnp.float32),
              jax.ShapeDtypeStruct((N_PAD, D), jnp.float32)),
    mesh=_mesh,
    compiler_params=_sc_params,
    scratch_types=[
        pltpu.VMEM_SHARED((N_PAD, D), jnp.float32),
        pltpu.VMEM((392, D), jnp.float32),
        pltpu.VMEM((CHUNK_ROWS * 128, D), jnp.float32),
        pltpu.VMEM((CHUNK_ROWS, 128), jnp.int32),
        pltpu.VMEM((CHUNK_ROWS, 128), jnp.int32),
        pltpu.VMEM((CHUNK_ROWS, 128), jnp.float32),
        pltpu.SemaphoreType.DMA,
        pltpu.SemaphoreType.DMA,
    ],
)


def _combine_body(a_ref, b_ref, o_ref):
    o_ref[...] = a_ref[...] + b_ref[...]


def _combine(a, b):
    m = N_PAD * D // 128
    blk = m // 8
    out = pl.pallas_call(
        _combine_body,
        grid=(8,),
        in_specs=[pl.BlockSpec((blk, 128), lambda i: (i, 0))] * 2,
        out_specs=pl.BlockSpec((blk, 128), lambda i: (i, 0)),
        out_shape=jax.ShapeDtypeStruct((m, 128), jnp.float32),
    )(a.reshape(m, 128), b.reshape(m, 128))
    return out.reshape(N_PAD, D)


IDX_ROWS = 3 * BATCH // 128
IDX_ROWS_PER_WORKER = IDX_ROWS // NW


def _bgather_body(t0, t1, t2, t3, idx_hbm, avg_o, g0_o,
                  ib, rb0, rb1, rb2, rb3, ob, gsem):
    c = lax.axis_index("c")
    s = lax.axis_index("s")
    wid = c * NS + s
    pltpu.sync_copy(idx_hbm.at[pl.ds(wid * IDX_ROWS_PER_WORKER, IDX_ROWS_PER_WORKER)], ib)
    quarter = jnp.full((16,), 0.25, jnp.float32)
    for j in range(IDX_ROWS_PER_WORKER):
        descs = [
            pltpu.async_copy(t.at[ib.at[j]], r, gsem)
            for t, r in ((t0, rb0), (t1, rb1), (t2, rb2), (t3, rb3))
        ]
        for d in descs:
            d.wait()

        def avg_body(r, _):
            for h in (0, 16):
                v = (rb0[r, pl.ds(h, 16)] + rb1[r, pl.ds(h, 16)]
                     + rb2[r, pl.ds(h, 16)] + rb3[r, pl.ds(h, 16)])
                ob[r, pl.ds(h, 16)] = v * quarter
            return 0

        lax.fori_loop(0, 128, avg_body, 0)
        base = wid * IDX_ROWS_PER_WORKER * 128 + j * 128
        pltpu.sync_copy(ob, avg_o.at[pl.ds(base, 128)])
        pltpu.sync_copy(rb0, g0_o.at[pl.ds(base, 128)])


_bgather = pl.kernel(
    _bgather_body,
    out_type=(jax.ShapeDtypeStruct((3 * BATCH, D), jnp.float32),
              jax.ShapeDtypeStruct((3 * BATCH, D), jnp.float32)),
    mesh=_mesh,
    compiler_params=_sc_params,
    scratch_types=[
        pltpu.VMEM((IDX_ROWS_PER_WORKER, 128), jnp.int32),
        pltpu.VMEM((128, D), jnp.float32),
        pltpu.VMEM((128, D), jnp.float32),
        pltpu.VMEM((128, D), jnp.float32),
        pltpu.VMEM((128, D), jnp.float32),
        pltpu.VMEM((128, D), jnp.float32),
        pltpu.SemaphoreType.DMA,
    ],
)


def _loss_body(ue, pe, ne, u0, p0, n0, o_ref):
    ps = jnp.sum(ue[...] * pe[...], axis=1, keepdims=True)
    ns = jnp.sum(ue[...] * ne[...], axis=1, keepdims=True)
    x = ns - ps
    sp = jnp.maximum(x, 0.0) + jnp.log1p(jnp.exp(-jnp.abs(x)))
    loss = jnp.sum(sp) / float(BATCH)
    reg = 0.5 * (jnp.sum(u0[...] ** 2) + jnp.sum(p0[...] ** 2)
                 + jnp.sum(n0[...] ** 2)) / float(BATCH)
    o_ref[...] = jnp.full((1, 1), loss + 1e-4 * reg, jnp.float32)


def _loss(ue, pe, ne, u0, p0, n0):
    return pl.pallas_call(
        _loss_body,
        out_shape=jax.ShapeDtypeStruct((1, 1), jnp.float32),
    )(ue, pe, ne, u0, p0, n0)


def kernel(user_emb, item_emb, edge_weight, edge_index, users, pos, neg):
    t0 = jnp.concatenate(
        [user_emb, item_emb,
         jnp.zeros((N_PAD - N_NODES, D), jnp.float32)], axis=0)
    pad = E_PAD - N_EDGES
    src = jnp.concatenate([edge_index[0], jnp.zeros((pad,), jnp.int32)]).reshape(E_ROWS, 128)
    dst = jnp.concatenate([edge_index[1], jnp.zeros((pad,), jnp.int32)]).reshape(E_ROWS, 128)
    w = jnp.concatenate([edge_weight, jnp.zeros((pad,), jnp.float32)]).reshape(E_ROWS, 128)

    tables = [t0]
    t = t0
    for _ in range(3):
        p0_, p1_ = _spmm(t, src, dst, w)
        t = _combine(p0_, p1_)
        tables.append(t)

    idx = jnp.concatenate([users, pos + N_USERS, neg + N_USERS]).reshape(IDX_ROWS, 128)
    avg, g0 = _bgather(tables[0], tables[1], tables[2], tables[3], idx)
    ue, pe, ne = avg[:BATCH], avg[BATCH:2 * BATCH], avg[2 * BATCH:]
    u0, pp0, nn0 = g0[:BATCH], g0[BATCH:2 * BATCH], g0[2 * BATCH:]
    out = _loss(ue, pe, ne, u0, pp0, nn0)
    return out[0, 0]


# speedup vs baseline: 11.2984x; 1.3629x over previous
import functools

import jax
import jax.numpy as jnp
from jax import lax
from jax.experimental import pallas as pl
from jax.experimental.pallas import tpu as pltpu
from jax.experimental.pallas import tpu_sc as plsc

N_USERS = 25000
N_NODES = 50000
N_EDGES = 1600000
D = 32
BATCH = 4096

NC = 2
NS = 16
NW = NC * NS

N_PAD = 50176
ROWS_PER_TILE = N_PAD // NS
E_PAD = 1638400
E_ROWS = E_PAD // 128
ROWS_PER_WORKER = E_ROWS // NW
CHUNK_ROWS = 2
N_CHUNKS = ROWS_PER_WORKER // CHUNK_ROWS

_mesh = plsc.VectorSubcoreMesh(core_axis_name="c", subcore_axis_name="s")


def _spmm_body(t_hbm, src_hbm, dst_hbm, w_hbm, out0, out1,
               acc, rb0, rb1, sb0, sb1, db0, db1, wb0, wb1,
               esem0, esem1, gsem0, gsem1, ssem0, ssem1):
    c = lax.axis_index("c")
    s = lax.axis_index("s")
    wid = c * NS + s
    rb = (rb0, rb1)
    sb = (sb0, sb1)
    db = (db0, db1)
    wb = (wb0, wb1)
    esem = (esem0, esem1)
    gsem = (gsem0, gsem1)
    ssem = (ssem0, ssem1)

    zero16 = jnp.zeros((16,), jnp.float32)
    zrows = rb0.shape[0]

    def zfill(i, _):
        rb0[i, pl.ds(0, 16)] = zero16
        rb0[i, pl.ds(16, 16)] = zero16
        return 0

    lax.fori_loop(0, zrows, zfill, 0)
    r0 = s * ROWS_PER_TILE
    nfull = ROWS_PER_TILE // zrows
    for r in range(nfull):
        pltpu.async_copy(rb0, acc.at[pl.ds(r0 + r * zrows, zrows)], esem0)
    rem = ROWS_PER_TILE - nfull * zrows
    if rem:
        pltpu.async_copy(rb0.at[pl.ds(0, rem)],
                         acc.at[pl.ds(r0 + nfull * zrows, rem)], esem0)
    for r in range(nfull):
        pltpu.make_async_copy(rb0, acc.at[pl.ds(r0 + r * zrows, zrows)], esem0).wait()
    if rem:
        pltpu.make_async_copy(rb0.at[pl.ds(0, rem)],
                              acc.at[pl.ds(r0 + nfull * zrows, rem)], esem0).wait()
    plsc.subcore_barrier()

    base_row = wid * ROWS_PER_WORKER

    def issue_edge_loads(g, b):
        row0 = base_row + g * CHUNK_ROWS
        pltpu.async_copy(src_hbm.at[pl.ds(row0, CHUNK_ROWS)], sb[b], esem[b])
        pltpu.async_copy(dst_hbm.at[pl.ds(row0, CHUNK_ROWS)], db[b], esem[b])
        pltpu.async_copy(w_hbm.at[pl.ds(row0, CHUNK_ROWS)], wb[b], esem[b])

    def wait_edge_loads(g, b):
        row0 = base_row + g * CHUNK_ROWS
        pltpu.make_async_copy(src_hbm.at[pl.ds(row0, CHUNK_ROWS)], sb[b], esem[b]).wait()
        pltpu.make_async_copy(dst_hbm.at[pl.ds(row0, CHUNK_ROWS)], db[b], esem[b]).wait()
        pltpu.make_async_copy(w_hbm.at[pl.ds(row0, CHUNK_ROWS)], wb[b], esem[b]).wait()

    def issue_gathers(b):
        for j in range(CHUNK_ROWS):
            pltpu.async_copy(t_hbm.at[sb[b].at[j]],
                             rb[b].at[pl.ds(j * 128, 128)], gsem[b])

    def wait_gathers(b):
        for j in range(CHUNK_ROWS):
            pltpu.make_async_copy(t_hbm.at[sb[b].at[j]],
                                  rb[b].at[pl.ds(j * 128, 128)], gsem[b]).wait()

    def issue_scatters(b):
        for j in range(CHUNK_ROWS):
            pltpu.async_copy(rb[b].at[pl.ds(j * 128, 128)],
                             acc.at[db[b].at[j]], ssem[b], add=True)

    def wait_scatters(b):
        for j in range(CHUNK_ROWS):
            pltpu.make_async_copy(rb[b].at[pl.ds(j * 128, 128)],
                                  acc.at[db[b].at[j]], ssem[b]).wait()

    def multiply(b):
        rbb = rb[b]
        wbb = wb[b]

        def mul_body(t, _):
            j = t >> 3
            i0 = (t & 7) * 16
            wvec = wbb[j, pl.ds(i0, 16)]
            e0 = t * 16
            for k in range(16):
                wv = jnp.full((16,), wvec[k], jnp.float32)
                rbb[e0 + k, pl.ds(0, 16)] = rbb[e0 + k, pl.ds(0, 16)] * wv
                rbb[e0 + k, pl.ds(16, 16)] = rbb[e0 + k, pl.ds(16, 16)] * wv
            return 0

        lax.fori_loop(0, CHUNK_ROWS * 8, mul_body, 0)

    def chunk_step(g, b):
        nb = 1 - b

        @pl.when(g >= 1)
        def _():
            wait_scatters(nb)

        @pl.when(g + 1 < N_CHUNKS)
        def _():
            issue_edge_loads(g + 1, nb)
            wait_edge_loads(g + 1, nb)
            issue_gathers(nb)

        wait_gathers(b)
        multiply(b)
        issue_scatters(b)

    issue_edge_loads(0, 0)
    wait_edge_loads(0, 0)
    issue_gathers(0)

    def loop_body(gg, _):
        chunk_step(gg * 2, 0)
        chunk_step(gg * 2 + 1, 1)
        return 0

    lax.fori_loop(0, N_CHUNKS // 2, loop_body, 0)
    wait_scatters(1)
    plsc.subcore_barrier()

    @pl.when(c == 0)
    def _():
        pltpu.sync_copy(acc.at[pl.ds(r0, ROWS_PER_TILE)],
                        out0.at[pl.ds(r0, ROWS_PER_TILE)])

    @pl.when(c == 1)
    def _():
        pltpu.sync_copy(acc.at[pl.ds(r0, ROWS_PER_TILE)],
                        out1.at[pl.ds(r0, ROWS_PER_TILE)])


_sc_params = pltpu.CompilerParams(use_tc_tiling_on_sc=False)

_spmm = pl.kernel(
    _spmm_body,
    out_type=(jax.ShapeDtypeStruct((N_PAD, D), jnp.float32),
              jax.ShapeDtypeStruct((N_PAD, D), jnp.float32)),
    mesh=_mesh,
    compiler_params=_sc_params,
    scratch_types=[
        pltpu.VMEM_SHARED((N_PAD, D), jnp.float32),
        pltpu.VMEM((CHUNK_ROWS * 128, D), jnp.float32),
        pltpu.VMEM((CHUNK_ROWS * 128, D), jnp.float32),
        pltpu.VMEM((CHUNK_ROWS, 128), jnp.int32),
        pltpu.VMEM((CHUNK_ROWS, 128), jnp.int32),
        pltpu.VMEM((CHUNK_ROWS, 128), jnp.int32),
        pltpu.VMEM((CHUNK_ROWS, 128), jnp.int32),
        pltpu.VMEM((CHUNK_ROWS, 128), jnp.float32),
        pltpu.VMEM((CHUNK_ROWS, 128), jnp.float32),
        pltpu.SemaphoreType.DMA,
        pltpu.SemaphoreType.DMA,
        pltpu.SemaphoreType.DMA,
        pltpu.SemaphoreType.DMA,
        pltpu.SemaphoreType.DMA,
        pltpu.SemaphoreType.DMA,
    ],
)


def _combine_body(a_ref, b_ref, o_ref):
    o_ref[...] = a_ref[...] + b_ref[...]


def _combine(a, b):
    m = N_PAD * D // 128
    blk = m // 8
    out = pl.pallas_call(
        _combine_body,
        grid=(8,),
        in_specs=[pl.BlockSpec((blk, 128), lambda i: (i, 0))] * 2,
        out_specs=pl.BlockSpec((blk, 128), lambda i: (i, 0)),
        out_shape=jax.ShapeDtypeStruct((m, 128), jnp.float32),
    )(a.reshape(m, 128), b.reshape(m, 128))
    return out.reshape(N_PAD, D)


IDX_ROWS = 3 * BATCH // 128
IDX_ROWS_PER_WORKER = IDX_ROWS // NW


def _bgather_body(t0, t1, t2, t3, idx_hbm, avg_o, g0_o,
                  ib, rb0, rb1, rb2, rb3, ob, gsem):
    c = lax.axis_index("c")
    s = lax.axis_index("s")
    wid = c * NS + s
    pltpu.sync_copy(idx_hbm.at[pl.ds(wid * IDX_ROWS_PER_WORKER, IDX_ROWS_PER_WORKER)], ib)
    quarter = jnp.full((16,), 0.25, jnp.float32)
    for j in range(IDX_ROWS_PER_WORKER):
        descs = [
            pltpu.async_copy(t.at[ib.at[j]], r, gsem)
            for t, r in ((t0, rb0), (t1, rb1), (t2, rb2), (t3, rb3))
        ]
        for d in descs:
            d.wait()

        def avg_body(r, _):
            for h in (0, 16):
                v = (rb0[r, pl.ds(h, 16)] + rb1[r, pl.ds(h, 16)]
                     + rb2[r, pl.ds(h, 16)] + rb3[r, pl.ds(h, 16)])
                ob[r, pl.ds(h, 16)] = v * quarter
            return 0

        lax.fori_loop(0, 128, avg_body, 0)
        base = wid * IDX_ROWS_PER_WORKER * 128 + j * 128
        pltpu.sync_copy(ob, avg_o.at[pl.ds(base, 128)])
        pltpu.sync_copy(rb0, g0_o.at[pl.ds(base, 128)])


_bgather = pl.kernel(
    _bgather_body,
    out_type=(jax.ShapeDtypeStruct((3 * BATCH, D), jnp.float32),
              jax.ShapeDtypeStruct((3 * BATCH, D), jnp.float32)),
    mesh=_mesh,
    compiler_params=_sc_params,
    scratch_types=[
        pltpu.VMEM((IDX_ROWS_PER_WORKER, 128), jnp.int32),
        pltpu.VMEM((128, D), jnp.float32),
        pltpu.VMEM((128, D), jnp.float32),
        pltpu.VMEM((128, D), jnp.float32),
        pltpu.VMEM((128, D), jnp.float32),
        pltpu.VMEM((128, D), jnp.float32),
        pltpu.SemaphoreType.DMA,
    ],
)


def _loss_body(ue, pe, ne, u0, p0, n0, o_ref):
    ps = jnp.sum(ue[...] * pe[...], axis=1, keepdims=True)
    ns = jnp.sum(ue[...] * ne[...], axis=1, keepdims=True)
    x = ns - ps
    sp = jnp.maximum(x, 0.0) + jnp.log1p(jnp.exp(-jnp.abs(x)))
    loss = jnp.sum(sp) / float(BATCH)
    reg = 0.5 * (jnp.sum(u0[...] ** 2) + jnp.sum(p0[...] ** 2)
                 + jnp.sum(n0[...] ** 2)) / float(BATCH)
    o_ref[...] = jnp.full((1, 1), loss + 1e-4 * reg, jnp.float32)


def _loss(ue, pe, ne, u0, p0, n0):
    return pl.pallas_call(
        _loss_body,
        out_shape=jax.ShapeDtypeStruct((1, 1), jnp.float32),
    )(ue, pe, ne, u0, p0, n0)


def kernel(user_emb, item_emb, edge_weight, edge_index, users, pos, neg):
    t0 = jnp.concatenate(
        [user_emb, item_emb,
         jnp.zeros((N_PAD - N_NODES, D), jnp.float32)], axis=0)
    pad = E_PAD - N_EDGES
    src = jnp.concatenate([edge_index[0], jnp.zeros((pad,), jnp.int32)]).reshape(E_ROWS, 128)
    dst = jnp.concatenate([edge_index[1], jnp.zeros((pad,), jnp.int32)]).reshape(E_ROWS, 128)
    w = jnp.concatenate([edge_weight, jnp.zeros((pad,), jnp.float32)]).reshape(E_ROWS, 128)

    tables = [t0]
    t = t0
    for _ in range(3):
        p0_, p1_ = _spmm(t, src, dst, w)
        t = _combine(p0_, p1_)
        tables.append(t)

    idx = jnp.concatenate([users, pos + N_USERS, neg + N_USERS]).reshape(IDX_ROWS, 128)
    avg, g0 = _bgather(tables[0], tables[1], tables[2], tables[3], idx)
    ue, pe, ne = avg[:BATCH], avg[BATCH:2 * BATCH], avg[2 * BATCH:]
    u0, pp0, nn0 = g0[:BATCH], g0[BATCH:2 * BATCH], g0[2 * BATCH:]
    out = _loss(ue, pe, ne, u0, pp0, nn0)
    return out[0, 0]


# speedup vs baseline: 16.3969x; 1.4513x over previous
import jax
import jax.numpy as jnp
from jax import lax
from jax.experimental import pallas as pl
from jax.experimental.pallas import tpu as pltpu
from jax.experimental.pallas import tpu_sc as plsc

N_USERS = 25000
N_NODES = 50000
N_EDGES = 1600000
D = 32
DH = 16
BATCH = 4096

NC = 2
NS = 16

N_PAD = 50176
ROWS_PER_TILE = N_PAD // NS
E_PAD = 1638400
E_ROWS = E_PAD // 128
ROWS_PER_TILE_E = E_ROWS // NS
CHUNK_ROWS = 16
N_CHUNKS = ROWS_PER_TILE_E // CHUNK_ROWS

_mesh = plsc.VectorSubcoreMesh(core_axis_name="c", subcore_axis_name="s")
_sc_params = pltpu.CompilerParams(use_tc_tiling_on_sc=False)


def _edge_pipeline(s, t_hbm, src_hbm, dst_hbm, w_hbm, out_hbm,
                   acc, rb, sb, db, wb, esem, gsem, ssem):
    base_row = s * ROWS_PER_TILE_E

    def issue_edge_loads(g, b):
        row0 = base_row + g * CHUNK_ROWS
        pltpu.async_copy(src_hbm.at[pl.ds(row0, CHUNK_ROWS)], sb[b], esem[b])
        pltpu.async_copy(dst_hbm.at[pl.ds(row0, CHUNK_ROWS)], db[b], esem[b])
        pltpu.async_copy(w_hbm.at[pl.ds(row0, CHUNK_ROWS)], wb[b], esem[b])

    def wait_edge_loads(g, b):
        row0 = base_row + g * CHUNK_ROWS
        pltpu.make_async_copy(src_hbm.at[pl.ds(row0, CHUNK_ROWS)], sb[b], esem[b]).wait()
        pltpu.make_async_copy(dst_hbm.at[pl.ds(row0, CHUNK_ROWS)], db[b], esem[b]).wait()
        pltpu.make_async_copy(w_hbm.at[pl.ds(row0, CHUNK_ROWS)], wb[b], esem[b]).wait()

    def issue_gathers(b):
        for j in range(CHUNK_ROWS):
            pltpu.async_copy(t_hbm.at[sb[b].at[j]],
                             rb[b].at[pl.ds(j * 128, 128)], gsem[b])

    def wait_gathers(b):
        for j in range(CHUNK_ROWS):
            pltpu.make_async_copy(t_hbm.at[sb[b].at[j]],
                                  rb[b].at[pl.ds(j * 128, 128)], gsem[b]).wait()

    def issue_scatters(b):
        for j in range(CHUNK_ROWS):
            pltpu.async_copy(rb[b].at[pl.ds(j * 128, 128)],
                             acc.at[db[b].at[j]], ssem[b], add=True)

    def wait_scatters(b):
        for j in range(CHUNK_ROWS):
            pltpu.make_async_copy(rb[b].at[pl.ds(j * 128, 128)],
                                  acc.at[db[b].at[j]], ssem[b]).wait()

    def multiply(b):
        rbb = rb[b]
        wbb = wb[b]

        def mul_body(t, _):
            j = t >> 3
            i0 = (t & 7) * 16
            wvec = wbb[j, pl.ds(i0, 16)]
            e0 = t * 16
            for k in range(16):
                wv = jnp.full((16,), wvec[k], jnp.float32)
                rbb[e0 + k, pl.ds(0, 16)] = rbb[e0 + k, pl.ds(0, 16)] * wv
            return 0

        lax.fori_loop(0, CHUNK_ROWS * 8, mul_body, 0)

    def chunk_step(g, b):
        nb = 1 - b

        @pl.when(g >= 1)
        def _():
            wait_scatters(nb)

        @pl.when(g + 1 < N_CHUNKS)
        def _():
            issue_edge_loads(g + 1, nb)
            wait_edge_loads(g + 1, nb)
            issue_gathers(nb)

        wait_gathers(b)
        multiply(b)
        issue_scatters(b)

    issue_edge_loads(0, 0)
    wait_edge_loads(0, 0)
    issue_gathers(0)

    def loop_body(gg, _):
        chunk_step(gg * 2, 0)
        chunk_step(gg * 2 + 1, 1)
        return 0

    lax.fori_loop(0, N_CHUNKS // 2, loop_body, 0)
    wait_scatters(1)
    plsc.subcore_barrier()

    r0 = s * ROWS_PER_TILE
    pltpu.sync_copy(acc.at[pl.ds(r0, ROWS_PER_TILE)],
                    out_hbm.at[pl.ds(r0, ROWS_PER_TILE)])


def _spmm_body(t_lo, t_hi, src_hbm, dst_hbm, w_hbm, out_lo, out_hi,
               acc, rb0, rb1, sb0, sb1, db0, db1, wb0, wb1,
               esem0, esem1, gsem0, gsem1, ssem0, ssem1):
    c = lax.axis_index("c")
    s = lax.axis_index("s")
    rb = (rb0, rb1)
    sb = (sb0, sb1)
    db = (db0, db1)
    wb = (wb0, wb1)
    esem = (esem0, esem1)
    gsem = (gsem0, gsem1)
    ssem = (ssem0, ssem1)

    zero16 = jnp.zeros((16,), jnp.float32)
    zrows = rb0.shape[0]

    def zfill(i, _):
        rb0[i, pl.ds(0, 16)] = zero16
        return 0

    lax.fori_loop(0, zrows, zfill, 0)
    r0 = s * ROWS_PER_TILE
    pltpu.async_copy(rb0, acc.at[pl.ds(r0, zrows)], esem0)
    rem = ROWS_PER_TILE - zrows
    pltpu.async_copy(rb0.at[pl.ds(0, rem)], acc.at[pl.ds(r0 + zrows, rem)], esem0)
    pltpu.make_async_copy(rb0, acc.at[pl.ds(r0, zrows)], esem0).wait()
    pltpu.make_async_copy(rb0.at[pl.ds(0, rem)],
                          acc.at[pl.ds(r0 + zrows, rem)], esem0).wait()
    plsc.subcore_barrier()

    @pl.when(c == 0)
    def _():
        _edge_pipeline(s, t_lo, src_hbm, dst_hbm, w_hbm, out_lo,
                       acc, rb, sb, db, wb, esem, gsem, ssem)

    @pl.when(c == 1)
    def _():
        _edge_pipeline(s, t_hi, src_hbm, dst_hbm, w_hbm, out_hi,
                       acc, rb, sb, db, wb, esem, gsem, ssem)


_spmm = pl.kernel(
    _spmm_body,
    out_type=(jax.ShapeDtypeStruct((N_PAD, DH), jnp.float32),
              jax.ShapeDtypeStruct((N_PAD, DH), jnp.float32)),
    mesh=_mesh,
    compiler_params=_sc_params,
    scratch_types=[
        pltpu.VMEM_SHARED((N_PAD, DH), jnp.float32),
        pltpu.VMEM((CHUNK_ROWS * 128, DH), jnp.float32),
        pltpu.VMEM((CHUNK_ROWS * 128, DH), jnp.float32),
        pltpu.VMEM((CHUNK_ROWS, 128), jnp.int32),
        pltpu.VMEM((CHUNK_ROWS, 128), jnp.int32),
        pltpu.VMEM((CHUNK_ROWS, 128), jnp.int32),
        pltpu.VMEM((CHUNK_ROWS, 128), jnp.int32),
        pltpu.VMEM((CHUNK_ROWS, 128), jnp.float32),
        pltpu.VMEM((CHUNK_ROWS, 128), jnp.float32),
        pltpu.SemaphoreType.DMA,
        pltpu.SemaphoreType.DMA,
        pltpu.SemaphoreType.DMA,
        pltpu.SemaphoreType.DMA,
        pltpu.SemaphoreType.DMA,
        pltpu.SemaphoreType.DMA,
    ],
)


IDX_ROWS = 3 * BATCH // 128
IDX_ROWS_PER_TILE = IDX_ROWS // NS


def _bgather_half(s, t0, t1, t2, t3, idx_hbm, avg_o, g0_o,
                  ib, rb0, rb1, rb2, rb3, ob, gsem):
    pltpu.sync_copy(idx_hbm.at[pl.ds(s * IDX_ROWS_PER_TILE, IDX_ROWS_PER_TILE)], ib)
    quarter = jnp.full((16,), 0.25, jnp.float32)
    for j in range(IDX_ROWS_PER_TILE):
        descs = [
            pltpu.async_copy(t.at[ib.at[j]], r, gsem)
            for t, r in ((t0, rb0), (t1, rb1), (t2, rb2), (t3, rb3))
        ]
        for d in descs:
            d.wait()

        def avg_body(r, _):
            v = (rb0[r, pl.ds(0, 16)] + rb1[r, pl.ds(0, 16)]
                 + rb2[r, pl.ds(0, 16)] + rb3[r, pl.ds(0, 16)])
            ob[r, pl.ds(0, 16)] = v * quarter
            return 0

        lax.fori_loop(0, 128, avg_body, 0)
        base = s * IDX_ROWS_PER_TILE * 128 + j * 128
        pltpu.sync_copy(ob, avg_o.at[pl.ds(base, 128)])
        pltpu.sync_copy(rb0, g0_o.at[pl.ds(base, 128)])


def _bgather_body(t0l, t1l, t2l, t3l, t0h, t1h, t2h, t3h, idx_hbm,
                  avg_lo, avg_hi, g0_lo, g0_hi,
                  ib, rb0, rb1, rb2, rb3, ob, gsem):
    c = lax.axis_index("c")
    s = lax.axis_index("s")

    @pl.when(c == 0)
    def _():
        _bgather_half(s, t0l, t1l, t2l, t3l, idx_hbm, avg_lo, g0_lo,
                      ib, rb0, rb1, rb2, rb3, ob, gsem)

    @pl.when(c == 1)
    def _():
        _bgather_half(s, t0h, t1h, t2h, t3h, idx_hbm, avg_hi, g0_hi,
                      ib, rb0, rb1, rb2, rb3, ob, gsem)


_bgather = pl.kernel(
    _bgather_body,
    out_type=(jax.ShapeDtypeStruct((3 * BATCH, DH), jnp.float32),
              jax.ShapeDtypeStruct((3 * BATCH, DH), jnp.float32),
              jax.ShapeDtypeStruct((3 * BATCH, DH), jnp.float32),
              jax.ShapeDtypeStruct((3 * BATCH, DH), jnp.float32)),
    mesh=_mesh,
    compiler_params=_sc_params,
    scratch_types=[
        pltpu.VMEM((IDX_ROWS_PER_TILE, 128), jnp.int32),
        pltpu.VMEM((128, DH), jnp.float32),
        pltpu.VMEM((128, DH), jnp.float32),
        pltpu.VMEM((128, DH), jnp.float32),
        pltpu.VMEM((128, DH), jnp.float32),
        pltpu.VMEM((128, DH), jnp.float32),
        pltpu.SemaphoreType.DMA,
    ],
)


def _loss_body(ue, pe, ne, u0, p0, n0, o_ref):
    ps = jnp.sum(ue[...] * pe[...], axis=1, keepdims=True)
    ns = jnp.sum(ue[...] * ne[...], axis=1, keepdims=True)
    x = ns - ps
    sp = jnp.maximum(x, 0.0) + jnp.log1p(jnp.exp(-jnp.abs(x)))
    loss = jnp.sum(sp) / float(BATCH)
    reg = 0.5 * (jnp.sum(u0[...] ** 2) + jnp.sum(p0[...] ** 2)
                 + jnp.sum(n0[...] ** 2)) / float(BATCH)
    o_ref[...] = jnp.full((1, 1), loss + 1e-4 * reg, jnp.float32)


def _loss(ue, pe, ne, u0, p0, n0):
    return pl.pallas_call(
        _loss_body,
        out_shape=jax.ShapeDtypeStruct((1, 1), jnp.float32),
    )(ue, pe, ne, u0, p0, n0)


def kernel(user_emb, item_emb, edge_weight, edge_index, users, pos, neg):
    t0 = jnp.concatenate(
        [user_emb, item_emb,
         jnp.zeros((N_PAD - N_NODES, D), jnp.float32)], axis=0)
    t0l, t0h = t0[:, :DH], t0[:, DH:]
    pad = E_PAD - N_EDGES
    src = jnp.concatenate([edge_index[0], jnp.zeros((pad,), jnp.int32)]).reshape(E_ROWS, 128)
    dst = jnp.concatenate([edge_index[1], jnp.zeros((pad,), jnp.int32)]).reshape(E_ROWS, 128)
    w = jnp.concatenate([edge_weight, jnp.zeros((pad,), jnp.float32)]).reshape(E_ROWS, 128)

    lo = [t0l]
    hi = [t0h]
    for _ in range(3):
        nlo, nhi = _spmm(lo[-1], hi[-1], src, dst, w)
        lo.append(nlo)
        hi.append(nhi)

    idx = jnp.concatenate([users, pos + N_USERS, neg + N_USERS]).reshape(IDX_ROWS, 128)
    avg_lo, avg_hi, g0_lo, g0_hi = _bgather(
        lo[0], lo[1], lo[2], lo[3], hi[0], hi[1], hi[2], hi[3], idx)
    avg = jnp.concatenate([avg_lo, avg_hi], axis=1)
    g0 = jnp.concatenate([g0_lo, g0_hi], axis=1)
    ue, pe, ne = avg[:BATCH], avg[BATCH:2 * BATCH], avg[2 * BATCH:]
    u0, pp0, nn0 = g0[:BATCH], g0[BATCH:2 * BATCH], g0[2 * BATCH:]
    out = _loss(ue, pe, ne, u0, pp0, nn0)
    return out[0, 0]


# speedup vs baseline: 16.4906x; 1.0057x over previous
import jax
import jax.numpy as jnp
from jax import lax
from jax.experimental import pallas as pl
from jax.experimental.pallas import tpu as pltpu
from jax.experimental.pallas import tpu_sc as plsc

N_USERS = 25000
N_NODES = 50000
N_EDGES = 1600000
D = 32
DH = 16
BATCH = 4096

NC = 2
NS = 16

N_PAD = 50176
ROWS_PER_TILE = N_PAD // NS
E_PAD = 1638400
E_ROWS = E_PAD // 128
ROWS_PER_TILE_E = E_ROWS // NS
CHUNK_ROWS = 16
N_CHUNKS = ROWS_PER_TILE_E // CHUNK_ROWS

_mesh = plsc.VectorSubcoreMesh(core_axis_name="c", subcore_axis_name="s")
_sc_params = pltpu.CompilerParams(use_tc_tiling_on_sc=False)


def _edge_pipeline(s, t_hbm, src_hbm, dst_hbm, w_hbm, out_hbm,
                   acc, rb, sb, db, wb, esem, gsem, ssem):
    base_e = s * ROWS_PER_TILE_E * 128
    ce = CHUNK_ROWS * 128

    def issue_edge_loads(g, b):
        e0 = base_e + g * ce
        pltpu.async_copy(src_hbm.at[pl.ds(e0, ce)], sb[b], esem[b])
        pltpu.async_copy(dst_hbm.at[pl.ds(e0, ce)], db[b], esem[b])
        pltpu.async_copy(w_hbm.at[pl.ds(e0, ce)], wb[b], esem[b])

    def wait_edge_loads(g, b):
        e0 = base_e + g * ce
        pltpu.make_async_copy(src_hbm.at[pl.ds(e0, ce)], sb[b], esem[b]).wait()
        pltpu.make_async_copy(dst_hbm.at[pl.ds(e0, ce)], db[b], esem[b]).wait()
        pltpu.make_async_copy(w_hbm.at[pl.ds(e0, ce)], wb[b], esem[b]).wait()

    def issue_gathers(b):
        pltpu.async_copy(t_hbm.at[sb[b]], rb[b], gsem[b])

    def wait_gathers(b):
        pltpu.make_async_copy(t_hbm.at[sb[b]], rb[b], gsem[b]).wait()

    def issue_scatters(b):
        pltpu.async_copy(rb[b], acc.at[db[b]], ssem[b], add=True)

    def wait_scatters(b):
        pltpu.make_async_copy(rb[b], acc.at[db[b]], ssem[b]).wait()

    lane_idx = [jnp.full((16,), k, jnp.int32) for k in range(16)]

    def multiply(b):
        rbb = rb[b]
        wbb = wb[b]

        def mul_body(t, _):
            wvec = wbb[pl.ds(t * 16, 16)]
            e0 = t * 16
            for k in range(16):
                wv = wvec.at[lane_idx[k]].get(mode="promise_in_bounds")
                rbb[e0 + k, pl.ds(0, 16)] = rbb[e0 + k, pl.ds(0, 16)] * wv
            return 0

        lax.fori_loop(0, CHUNK_ROWS * 8, mul_body, 0)

    def chunk_step(g, b):
        nb = 1 - b

        @pl.when(g >= 1)
        def _():
            wait_scatters(nb)

        @pl.when(g + 1 < N_CHUNKS)
        def _():
            issue_edge_loads(g + 1, nb)
            wait_edge_loads(g + 1, nb)
            issue_gathers(nb)

        wait_gathers(b)
        multiply(b)
        issue_scatters(b)

    issue_edge_loads(0, 0)
    wait_edge_loads(0, 0)
    issue_gathers(0)

    def loop_body(gg, _):
        chunk_step(gg * 2, 0)
        chunk_step(gg * 2 + 1, 1)
        return 0

    lax.fori_loop(0, N_CHUNKS // 2, loop_body, 0)
    wait_scatters(1)
    plsc.subcore_barrier()

    r0 = s * ROWS_PER_TILE
    pltpu.sync_copy(acc.at[pl.ds(r0, ROWS_PER_TILE)],
                    out_hbm.at[pl.ds(r0, ROWS_PER_TILE)])


def _spmm_body(t_lo, t_hi, src_hbm, dst_hbm, w_hbm, out_lo, out_hi,
               acc, rb0, rb1, sb0, sb1, db0, db1, wb0, wb1,
               esem0, esem1, gsem0, gsem1, ssem0, ssem1):
    c = lax.axis_index("c")
    s = lax.axis_index("s")
    rb = (rb0, rb1)
    sb = (sb0, sb1)
    db = (db0, db1)
    wb = (wb0, wb1)
    esem = (esem0, esem1)
    gsem = (gsem0, gsem1)
    ssem = (ssem0, ssem1)

    zero16 = jnp.zeros((16,), jnp.float32)
    zrows = rb0.shape[0]

    def zfill(i, _):
        rb0[i, pl.ds(0, 16)] = zero16
        return 0

    lax.fori_loop(0, zrows, zfill, 0)
    r0 = s * ROWS_PER_TILE
    pltpu.async_copy(rb0, acc.at[pl.ds(r0, zrows)], esem0)
    rem = ROWS_PER_TILE - zrows
    pltpu.async_copy(rb0.at[pl.ds(0, rem)], acc.at[pl.ds(r0 + zrows, rem)], esem0)
    pltpu.make_async_copy(rb0, acc.at[pl.ds(r0, zrows)], esem0).wait()
    pltpu.make_async_copy(rb0.at[pl.ds(0, rem)],
                          acc.at[pl.ds(r0 + zrows, rem)], esem0).wait()
    plsc.subcore_barrier()

    @pl.when(c == 0)
    def _():
        _edge_pipeline(s, t_lo, src_hbm, dst_hbm, w_hbm, out_lo,
                       acc, rb, sb, db, wb, esem, gsem, ssem)

    @pl.when(c == 1)
    def _():
        _edge_pipeline(s, t_hi, src_hbm, dst_hbm, w_hbm, out_hi,
                       acc, rb, sb, db, wb, esem, gsem, ssem)


_spmm = pl.kernel(
    _spmm_body,
    out_type=(jax.ShapeDtypeStruct((N_PAD, DH), jnp.float32),
              jax.ShapeDtypeStruct((N_PAD, DH), jnp.float32)),
    mesh=_mesh,
    compiler_params=_sc_params,
    scratch_types=[
        pltpu.VMEM_SHARED((N_PAD, DH), jnp.float32),
        pltpu.VMEM((CHUNK_ROWS * 128, DH), jnp.float32),
        pltpu.VMEM((CHUNK_ROWS * 128, DH), jnp.float32),
        pltpu.VMEM((CHUNK_ROWS * 128,), jnp.int32),
        pltpu.VMEM((CHUNK_ROWS * 128,), jnp.int32),
        pltpu.VMEM((CHUNK_ROWS * 128,), jnp.int32),
        pltpu.VMEM((CHUNK_ROWS * 128,), jnp.int32),
        pltpu.VMEM((CHUNK_ROWS * 128,), jnp.float32),
        pltpu.VMEM((CHUNK_ROWS * 128,), jnp.float32),
        pltpu.SemaphoreType.DMA,
        pltpu.SemaphoreType.DMA,
        pltpu.SemaphoreType.DMA,
        pltpu.SemaphoreType.DMA,
        pltpu.SemaphoreType.DMA,
        pltpu.SemaphoreType.DMA,
    ],
)


IDX_ROWS = 3 * BATCH // 128
IDX_ROWS_PER_TILE = IDX_ROWS // NS


def _bgather_half(s, t0, t1, t2, t3, idx_hbm, avg_o, g0_o,
                  ib, rb0, rb1, rb2, rb3, ob, gsem):
    pltpu.sync_copy(idx_hbm.at[pl.ds(s * IDX_ROWS_PER_TILE, IDX_ROWS_PER_TILE)], ib)
    quarter = jnp.full((16,), 0.25, jnp.float32)
    for j in range(IDX_ROWS_PER_TILE):
        descs = [
            pltpu.async_copy(t.at[ib.at[j]], r, gsem)
            for t, r in ((t0, rb0), (t1, rb1), (t2, rb2), (t3, rb3))
        ]
        for d in descs:
            d.wait()

        def avg_body(r, _):
            v = (rb0[r, pl.ds(0, 16)] + rb1[r, pl.ds(0, 16)]
                 + rb2[r, pl.ds(0, 16)] + rb3[r, pl.ds(0, 16)])
            ob[r, pl.ds(0, 16)] = v * quarter
            return 0

        lax.fori_loop(0, 128, avg_body, 0)
        base = s * IDX_ROWS_PER_TILE * 128 + j * 128
        pltpu.sync_copy(ob, avg_o.at[pl.ds(base, 128)])
        pltpu.sync_copy(rb0, g0_o.at[pl.ds(base, 128)])


def _bgather_body(t0l, t1l, t2l, t3l, t0h, t1h, t2h, t3h, idx_hbm,
                  avg_lo, avg_hi, g0_lo, g0_hi,
                  ib, rb0, rb1, rb2, rb3, ob, gsem):
    c = lax.axis_index("c")
    s = lax.axis_index("s")

    @pl.when(c == 0)
    def _():
        _bgather_half(s, t0l, t1l, t2l, t3l, idx_hbm, avg_lo, g0_lo,
                      ib, rb0, rb1, rb2, rb3, ob, gsem)

    @pl.when(c == 1)
    def _():
        _bgather_half(s, t0h, t1h, t2h, t3h, idx_hbm, avg_hi, g0_hi,
                      ib, rb0, rb1, rb2, rb3, ob, gsem)


_bgather = pl.kernel(
    _bgather_body,
    out_type=(jax.ShapeDtypeStruct((3 * BATCH, DH), jnp.float32),
              jax.ShapeDtypeStruct((3 * BATCH, DH), jnp.float32),
              jax.ShapeDtypeStruct((3 * BATCH, DH), jnp.float32),
              jax.ShapeDtypeStruct((3 * BATCH, DH), jnp.float32)),
    mesh=_mesh,
    compiler_params=_sc_params,
    scratch_types=[
        pltpu.VMEM((IDX_ROWS_PER_TILE, 128), jnp.int32),
        pltpu.VMEM((128, DH), jnp.float32),
        pltpu.VMEM((128, DH), jnp.float32),
        pltpu.VMEM((128, DH), jnp.float32),
        pltpu.VMEM((128, DH), jnp.float32),
        pltpu.VMEM((128, DH), jnp.float32),
        pltpu.SemaphoreType.DMA,
    ],
)


def _loss_body(ue, pe, ne, u0, p0, n0, o_ref):
    ps = jnp.sum(ue[...] * pe[...], axis=1, keepdims=True)
    ns = jnp.sum(ue[...] * ne[...], axis=1, keepdims=True)
    x = ns - ps
    sp = jnp.maximum(x, 0.0) + jnp.log1p(jnp.exp(-jnp.abs(x)))
    loss = jnp.sum(sp) / float(BATCH)
    reg = 0.5 * (jnp.sum(u0[...] ** 2) + jnp.sum(p0[...] ** 2)
                 + jnp.sum(n0[...] ** 2)) / float(BATCH)
    o_ref[...] = jnp.full((1, 1), loss + 1e-4 * reg, jnp.float32)


def _loss(ue, pe, ne, u0, p0, n0):
    return pl.pallas_call(
        _loss_body,
        out_shape=jax.ShapeDtypeStruct((1, 1), jnp.float32),
    )(ue, pe, ne, u0, p0, n0)


def kernel(user_emb, item_emb, edge_weight, edge_index, users, pos, neg):
    t0 = jnp.concatenate(
        [user_emb, item_emb,
         jnp.zeros((N_PAD - N_NODES, D), jnp.float32)], axis=0)
    t0l, t0h = t0[:, :DH], t0[:, DH:]
    pad = E_PAD - N_EDGES
    src = jnp.concatenate([edge_index[0], jnp.zeros((pad,), jnp.int32)])
    dst = jnp.concatenate([edge_index[1], jnp.zeros((pad,), jnp.int32)])
    w = jnp.concatenate([edge_weight, jnp.zeros((pad,), jnp.float32)])

    lo = [t0l]
    hi = [t0h]
    for _ in range(3):
        nlo, nhi = _spmm(lo[-1], hi[-1], src, dst, w)
        lo.append(nlo)
        hi.append(nhi)

    idx = jnp.concatenate([users, pos + N_USERS, neg + N_USERS]).reshape(IDX_ROWS, 128)
    avg_lo, avg_hi, g0_lo, g0_hi = _bgather(
        lo[0], lo[1], lo[2], lo[3], hi[0], hi[1], hi[2], hi[3], idx)
    avg = jnp.concatenate([avg_lo, avg_hi], axis=1)
    g0 = jnp.concatenate([g0_lo, g0_hi], axis=1)
    ue, pe, ne = avg[:BATCH], avg[BATCH:2 * BATCH], avg[2 * BATCH:]
    u0, pp0, nn0 = g0[:BATCH], g0[BATCH:2 * BATCH], g0[2 * BATCH:]
    out = _loss(ue, pe, ne, u0, pp0, nn0)
    return out[0, 0]


# speedup vs baseline: 25.4353x; 1.5424x over previous
import jax
import jax.numpy as jnp
from jax import lax
from jax.experimental import pallas as pl
from jax.experimental.pallas import tpu as pltpu
from jax.experimental.pallas import tpu_sc as plsc

N_USERS = 25000
N_NODES = 50000
N_EDGES = 1600000
D = 32
DH = 16
BATCH = 4096

NC = 2
NS = 16

N_PAD = 50176
ROWS_PER_TILE = N_PAD // NS
E_PAD = 1638400
E_ROWS = E_PAD // 128
ROWS_PER_TILE_E = E_ROWS // NS
CHUNK_ROWS = 5
N_CHUNKS = ROWS_PER_TILE_E // CHUNK_ROWS

_mesh = plsc.VectorSubcoreMesh(core_axis_name="c", subcore_axis_name="s")
_sc_params = pltpu.CompilerParams(use_tc_tiling_on_sc=False)


def _edge_pipeline(s, tbl, src_hbm, dst_hbm, w_hbm, out_hbm,
                   acc, rb, sb, db, wb, esem, gsem, ssem):
    base_e = s * ROWS_PER_TILE_E * 128
    ce = CHUNK_ROWS * 128

    def issue_edge_loads(g, e):
        e0 = base_e + g * ce
        pltpu.async_copy(src_hbm.at[pl.ds(e0, ce)], sb[e], esem[e])
        pltpu.async_copy(dst_hbm.at[pl.ds(e0, ce)], db[e], esem[e])
        pltpu.async_copy(w_hbm.at[pl.ds(e0, ce)], wb[e], esem[e])

    def wait_edge_loads(g, e):
        e0 = base_e + g * ce
        pltpu.make_async_copy(src_hbm.at[pl.ds(e0, ce)], sb[e], esem[e]).wait()
        pltpu.make_async_copy(dst_hbm.at[pl.ds(e0, ce)], db[e], esem[e]).wait()
        pltpu.make_async_copy(w_hbm.at[pl.ds(e0, ce)], wb[e], esem[e]).wait()

    def issue_gathers(b, e):
        pltpu.async_copy(tbl.at[sb[e]], rb[b], gsem[b])

    def wait_gathers(b, e):
        pltpu.make_async_copy(tbl.at[sb[e]], rb[b], gsem[b]).wait()

    def issue_scatters(b, e):
        pltpu.async_copy(rb[b], acc.at[db[e]], ssem[b], add=True)

    def wait_scatters(b, e):
        pltpu.make_async_copy(rb[b], acc.at[db[e]], ssem[b]).wait()

    lane_idx = [jnp.full((16,), k, jnp.int32) for k in range(16)]

    def multiply(b, e):
        rbb = rb[b]
        wbb = wb[e]

        def mul_body(t, _):
            wvec = wbb[pl.ds(t * 16, 16)]
            e0 = t * 16
            for k in range(16):
                wv = wvec.at[lane_idx[k]].get(mode="promise_in_bounds")
                rbb[e0 + k, pl.ds(0, 16)] = rbb[e0 + k, pl.ds(0, 16)] * wv
            return 0

        lax.fori_loop(0, CHUNK_ROWS * 8, mul_body, 0)

    def chunk_step(g, u):
        b = u % 2
        nb = 1 - b

        @pl.when(g >= 1)
        def _():
            wait_scatters(nb, (u + 3) % 4)

        @pl.when(g + 2 < N_CHUNKS)
        def _():
            issue_edge_loads(g + 2, (u + 2) % 4)

        @pl.when(g + 1 < N_CHUNKS)
        def _():
            wait_edge_loads(g + 1, (u + 1) % 4)
            issue_gathers(nb, (u + 1) % 4)

        wait_gathers(b, u)
        multiply(b, u)
        issue_scatters(b, u)

    issue_edge_loads(0, 0)
    issue_edge_loads(1, 1)
    wait_edge_loads(0, 0)
    issue_gathers(0, 0)

    def loop_body(gg, _):
        for u in range(4):
            chunk_step(gg * 4 + u, u)
        return 0

    lax.fori_loop(0, N_CHUNKS // 4, loop_body, 0)
    wait_scatters(1, 3)
    plsc.subcore_barrier()

    r0 = s * ROWS_PER_TILE
    pltpu.sync_copy(acc.at[pl.ds(r0, ROWS_PER_TILE)],
                    out_hbm.at[pl.ds(r0, ROWS_PER_TILE)])


def _spmm_body(t_lo, t_hi, src_hbm, dst_hbm, w_hbm, out_lo, out_hi,
               acc, tbl, rb0, rb1, sb0, sb1, sb2, sb3, db0, db1, db2, db3,
               wb0, wb1, wb2, wb3,
               esem0, esem1, esem2, esem3, gsem0, gsem1, ssem0, ssem1):
    c = lax.axis_index("c")
    s = lax.axis_index("s")
    rb = (rb0, rb1)
    sb = (sb0, sb1, sb2, sb3)
    db = (db0, db1, db2, db3)
    wb = (wb0, wb1, wb2, wb3)
    esem = (esem0, esem1, esem2, esem3)
    gsem = (gsem0, gsem1)
    ssem = (ssem0, ssem1)

    r0 = s * ROWS_PER_TILE

    @pl.when(c == 0)
    def _():
        pltpu.async_copy(t_lo.at[pl.ds(r0, ROWS_PER_TILE)],
                         tbl.at[pl.ds(r0, ROWS_PER_TILE)], gsem0)

    @pl.when(c == 1)
    def _():
        pltpu.async_copy(t_hi.at[pl.ds(r0, ROWS_PER_TILE)],
                         tbl.at[pl.ds(r0, ROWS_PER_TILE)], gsem0)

    zero16 = jnp.zeros((16,), jnp.float32)
    zrows = rb0.shape[0]

    def zfill(i, _):
        rb0[i, pl.ds(0, 16)] = zero16
        return 0

    lax.fori_loop(0, zrows, zfill, 0)
    nfull = ROWS_PER_TILE // zrows
    for r in range(nfull):
        pltpu.async_copy(rb0, acc.at[pl.ds(r0 + r * zrows, zrows)], esem0)
    rem = ROWS_PER_TILE - nfull * zrows
    if rem:
        pltpu.async_copy(rb0.at[pl.ds(0, rem)],
                         acc.at[pl.ds(r0 + nfull * zrows, rem)], esem0)
    for r in range(nfull):
        pltpu.make_async_copy(rb0, acc.at[pl.ds(r0 + r * zrows, zrows)], esem0).wait()
    if rem:
        pltpu.make_async_copy(rb0.at[pl.ds(0, rem)],
                              acc.at[pl.ds(r0 + nfull * zrows, rem)], esem0).wait()

    @pl.when(c == 0)
    def _():
        pltpu.make_async_copy(t_lo.at[pl.ds(r0, ROWS_PER_TILE)],
                              tbl.at[pl.ds(r0, ROWS_PER_TILE)], gsem0).wait()

    @pl.when(c == 1)
    def _():
        pltpu.make_async_copy(t_hi.at[pl.ds(r0, ROWS_PER_TILE)],
                              tbl.at[pl.ds(r0, ROWS_PER_TILE)], gsem0).wait()

    plsc.subcore_barrier()

    @pl.when(c == 0)
    def _():
        _edge_pipeline(s, tbl, src_hbm, dst_hbm, w_hbm, out_lo,
                       acc, rb, sb, db, wb, esem, gsem, ssem)

    @pl.when(c == 1)
    def _():
        _edge_pipeline(s, tbl, src_hbm, dst_hbm, w_hbm, out_hi,
                       acc, rb, sb, db, wb, esem, gsem, ssem)


_spmm = pl.kernel(
    _spmm_body,
    out_type=(jax.ShapeDtypeStruct((N_PAD, DH), jnp.float32),
              jax.ShapeDtypeStruct((N_PAD, DH), jnp.float32)),
    mesh=_mesh,
    compiler_params=_sc_params,
    scratch_types=[
        pltpu.VMEM_SHARED((N_PAD, DH), jnp.float32),
        pltpu.VMEM_SHARED((N_PAD, DH), jnp.float32),
        pltpu.VMEM((CHUNK_ROWS * 128, DH), jnp.float32),
        pltpu.VMEM((CHUNK_ROWS * 128, DH), jnp.float32),
        pltpu.VMEM((CHUNK_ROWS * 128,), jnp.int32),
        pltpu.VMEM((CHUNK_ROWS * 128,), jnp.int32),
        pltpu.VMEM((CHUNK_ROWS * 128,), jnp.int32),
        pltpu.VMEM((CHUNK_ROWS * 128,), jnp.int32),
        pltpu.VMEM((CHUNK_ROWS * 128,), jnp.int32),
        pltpu.VMEM((CHUNK_ROWS * 128,), jnp.int32),
        pltpu.VMEM((CHUNK_ROWS * 128,), jnp.int32),
        pltpu.VMEM((CHUNK_ROWS * 128,), jnp.int32),
        pltpu.VMEM((CHUNK_ROWS * 128,), jnp.float32),
        pltpu.VMEM((CHUNK_ROWS * 128,), jnp.float32),
        pltpu.VMEM((CHUNK_ROWS * 128,), jnp.float32),
        pltpu.VMEM((CHUNK_ROWS * 128,), jnp.float32),
        pltpu.SemaphoreType.DMA,
        pltpu.SemaphoreType.DMA,
        pltpu.SemaphoreType.DMA,
        pltpu.SemaphoreType.DMA,
        pltpu.SemaphoreType.DMA,
        pltpu.SemaphoreType.DMA,
        pltpu.SemaphoreType.DMA,
        pltpu.SemaphoreType.DMA,
    ],
)


IDX_ROWS = 3 * BATCH // 128
IDX_ROWS_PER_TILE = IDX_ROWS // NS


def _bgather_half(s, t0, t1, t2, t3, idx_hbm, avg_o, g0_o,
                  ib, rb0, rb1, rb2, rb3, ob, gsem):
    pltpu.sync_copy(idx_hbm.at[pl.ds(s * IDX_ROWS_PER_TILE, IDX_ROWS_PER_TILE)], ib)
    quarter = jnp.full((16,), 0.25, jnp.float32)
    for j in range(IDX_ROWS_PER_TILE):
        descs = [
            pltpu.async_copy(t.at[ib.at[j]], r, gsem)
            for t, r in ((t0, rb0), (t1, rb1), (t2, rb2), (t3, rb3))
        ]
        for d in descs:
            d.wait()

        def avg_body(r, _):
            v = (rb0[r, pl.ds(0, 16)] + rb1[r, pl.ds(0, 16)]
                 + rb2[r, pl.ds(0, 16)] + rb3[r, pl.ds(0, 16)])
            ob[r, pl.ds(0, 16)] = v * quarter
            return 0

        lax.fori_loop(0, 128, avg_body, 0)
        base = s * IDX_ROWS_PER_TILE * 128 + j * 128
        pltpu.sync_copy(ob, avg_o.at[pl.ds(base, 128)])
        pltpu.sync_copy(rb0, g0_o.at[pl.ds(base, 128)])


def _bgather_body(t0l, t1l, t2l, t3l, t0h, t1h, t2h, t3h, idx_hbm,
                  avg_lo, avg_hi, g0_lo, g0_hi,
                  ib, rb0, rb1, rb2, rb3, ob, gsem):
    c = lax.axis_index("c")
    s = lax.axis_index("s")

    @pl.when(c == 0)
    def _():
        _bgather_half(s, t0l, t1l, t2l, t3l, idx_hbm, avg_lo, g0_lo,
                      ib, rb0, rb1, rb2, rb3, ob, gsem)

    @pl.when(c == 1)
    def _():
        _bgather_half(s, t0h, t1h, t2h, t3h, idx_hbm, avg_hi, g0_hi,
                      ib, rb0, rb1, rb2, rb3, ob, gsem)


_bgather = pl.kernel(
    _bgather_body,
    out_type=(jax.ShapeDtypeStruct((3 * BATCH, DH), jnp.float32),
              jax.ShapeDtypeStruct((3 * BATCH, DH), jnp.float32),
              jax.ShapeDtypeStruct((3 * BATCH, DH), jnp.float32),
              jax.ShapeDtypeStruct((3 * BATCH, DH), jnp.float32)),
    mesh=_mesh,
    compiler_params=_sc_params,
    scratch_types=[
        pltpu.VMEM((IDX_ROWS_PER_TILE, 128), jnp.int32),
        pltpu.VMEM((128, DH), jnp.float32),
        pltpu.VMEM((128, DH), jnp.float32),
        pltpu.VMEM((128, DH), jnp.float32),
        pltpu.VMEM((128, DH), jnp.float32),
        pltpu.VMEM((128, DH), jnp.float32),
        pltpu.SemaphoreType.DMA,
    ],
)


def _loss_body(ue, pe, ne, u0, p0, n0, o_ref):
    ps = jnp.sum(ue[...] * pe[...], axis=1, keepdims=True)
    ns = jnp.sum(ue[...] * ne[...], axis=1, keepdims=True)
    x = ns - ps
    sp = jnp.maximum(x, 0.0) + jnp.log1p(jnp.exp(-jnp.abs(x)))
    loss = jnp.sum(sp) / float(BATCH)
    reg = 0.5 * (jnp.sum(u0[...] ** 2) + jnp.sum(p0[...] ** 2)
                 + jnp.sum(n0[...] ** 2)) / float(BATCH)
    o_ref[...] = jnp.full((1, 1), loss + 1e-4 * reg, jnp.float32)


def _loss(ue, pe, ne, u0, p0, n0):
    return pl.pallas_call(
        _loss_body,
        out_shape=jax.ShapeDtypeStruct((1, 1), jnp.float32),
    )(ue, pe, ne, u0, p0, n0)


def kernel(user_emb, item_emb, edge_weight, edge_index, users, pos, neg):
    t0 = jnp.concatenate(
        [user_emb, item_emb,
         jnp.zeros((N_PAD - N_NODES, D), jnp.float32)], axis=0)
    t0l, t0h = t0[:, :DH], t0[:, DH:]
    pad = E_PAD - N_EDGES
    src = jnp.concatenate([edge_index[0], jnp.zeros((pad,), jnp.int32)])
    dst = jnp.concatenate([edge_index[1], jnp.zeros((pad,), jnp.int32)])
    w = jnp.concatenate([edge_weight, jnp.zeros((pad,), jnp.float32)])

    lo = [t0l]
    hi = [t0h]
    for _ in range(3):
        nlo, nhi = _spmm(lo[-1], hi[-1], src, dst, w)
        lo.append(nlo)
        hi.append(nhi)

    idx = jnp.concatenate([users, pos + N_USERS, neg + N_USERS]).reshape(IDX_ROWS, 128)
    avg_lo, avg_hi, g0_lo, g0_hi = _bgather(
        lo[0], lo[1], lo[2], lo[3], hi[0], hi[1], hi[2], hi[3], idx)
    avg = jnp.concatenate([avg_lo, avg_hi], axis=1)
    g0 = jnp.concatenate([g0_lo, g0_hi], axis=1)
    ue, pe, ne = avg[:BATCH], avg[BATCH:2 * BATCH], avg[2 * BATCH:]
    u0, pp0, nn0 = g0[:BATCH], g0[BATCH:2 * BATCH], g0[2 * BATCH:]
    out = _loss(ue, pe, ne, u0, pp0, nn0)
    return out[0, 0]


# speedup vs baseline: 26.3949x; 1.0377x over previous
import jax
import jax.numpy as jnp
from jax import lax
from jax.experimental import pallas as pl
from jax.experimental.pallas import tpu as pltpu
from jax.experimental.pallas import tpu_sc as plsc

N_USERS = 25000
N_NODES = 50000
N_EDGES = 1600000
D = 32
DH = 16
BATCH = 4096

NC = 2
NS = 16

N_PAD = 50176
ROWS_PER_TILE = N_PAD // NS
E_PAD = 1638400
E_ROWS = E_PAD // 128
ROWS_PER_TILE_E = E_ROWS // NS
CHUNK_ROWS = 5
N_CHUNKS = ROWS_PER_TILE_E // CHUNK_ROWS

B3 = 3 * BATCH
BPT = B3 // NS

_mesh = plsc.VectorSubcoreMesh(core_axis_name="c", subcore_axis_name="s")
_sc_params = pltpu.CompilerParams(use_tc_tiling_on_sc=False)


def _edge_pipeline(s, tbl, acc, src_hbm, dst_hbm, w_hbm,
                   rb, sb, db, wb, esem, gsem, ssem):
    base_e = s * ROWS_PER_TILE_E * 128
    ce = CHUNK_ROWS * 128

    def issue_edge_loads(g, e):
        e0 = base_e + g * ce
        pltpu.async_copy(src_hbm.at[pl.ds(e0, ce)], sb[e], esem[e])
        pltpu.async_copy(dst_hbm.at[pl.ds(e0, ce)], db[e], esem[e])
        pltpu.async_copy(w_hbm.at[pl.ds(e0, ce)], wb[e], esem[e])

    def wait_edge_loads(g, e):
        e0 = base_e + g * ce
        pltpu.make_async_copy(src_hbm.at[pl.ds(e0, ce)], sb[e], esem[e]).wait()
        pltpu.make_async_copy(dst_hbm.at[pl.ds(e0, ce)], db[e], esem[e]).wait()
        pltpu.make_async_copy(w_hbm.at[pl.ds(e0, ce)], wb[e], esem[e]).wait()

    def issue_gathers(b, e):
        pltpu.async_copy(tbl.at[sb[e]], rb[b], gsem[b])

    def wait_gathers(b, e):
        pltpu.make_async_copy(tbl.at[sb[e]], rb[b], gsem[b]).wait()

    def issue_scatters(b, e):
        pltpu.async_copy(rb[b], acc.at[db[e]], ssem[b], add=True)

    def wait_scatters(b, e):
        pltpu.make_async_copy(rb[b], acc.at[db[e]], ssem[b]).wait()

    lane_idx = [jnp.full((16,), k, jnp.int32) for k in range(16)]

    def multiply(b, e):
        rbb = rb[b]
        wbb = wb[e]

        def mul_body(t, _):
            wvec = wbb[pl.ds(t * 16, 16)]
            e0 = t * 16
            for k in range(16):
                wv = wvec.at[lane_idx[k]].get(mode="promise_in_bounds")
                rbb[e0 + k, pl.ds(0, 16)] = rbb[e0 + k, pl.ds(0, 16)] * wv
            return 0

        lax.fori_loop(0, CHUNK_ROWS * 8, mul_body, 0)

    def chunk_step(g, u):
        b = u % 2
        nb = 1 - b

        @pl.when(g >= 1)
        def _():
            wait_scatters(nb, (u + 3) % 4)

        @pl.when(g + 2 < N_CHUNKS)
        def _():
            issue_edge_loads(g + 2, (u + 2) % 4)

        @pl.when(g + 1 < N_CHUNKS)
        def _():
            wait_edge_loads(g + 1, (u + 1) % 4)
            issue_gathers(nb, (u + 1) % 4)

        wait_gathers(b, u)
        multiply(b, u)
        issue_scatters(b, u)

    issue_edge_loads(0, 0)
    issue_edge_loads(1, 1)
    wait_edge_loads(0, 0)
    issue_gathers(0, 0)

    def loop_body(gg, _):
        for u in range(4):
            chunk_step(gg * 4 + u, u)
        return 0

    lax.fori_loop(0, N_CHUNKS // 4, loop_body, 0)
    wait_scatters(1, 3)
    plsc.subcore_barrier()


def _mega_body(t_lo, t_hi, src_hbm, dst_hbm, w_hbm, idx_hbm, out_lo, out_hi,
               ta, tb, rb0, rb1, sb0, sb1, sb2, sb3, db0, db1, db2, db3,
               wb0, wb1, wb2, wb3, ib640, ib128,
               esem0, esem1, esem2, esem3, gsem0, gsem1, ssem0, ssem1):
    c = lax.axis_index("c")
    s = lax.axis_index("s")
    rb = (rb0, rb1)
    sb = (sb0, sb1, sb2, sb3)
    db = (db0, db1, db2, db3)
    wb = (wb0, wb1, wb2, wb3)
    esem = (esem0, esem1, esem2, esem3)
    gsem = (gsem0, gsem1)
    ssem = (ssem0, ssem1)
    r0 = s * ROWS_PER_TILE
    zero16 = jnp.zeros((16,), jnp.float32)
    zrows = rb0.shape[0]

    def zero_fill_rb0():
        def zfill(i, _):
            rb0[i, pl.ds(0, 16)] = zero16
            return 0

        lax.fori_loop(0, zrows, zfill, 0)

    def zero_acc(accr):
        zero_fill_rb0()
        nfull = ROWS_PER_TILE // zrows
        for r in range(nfull):
            pltpu.async_copy(rb0, accr.at[pl.ds(r0 + r * zrows, zrows)], esem0)
        rem = ROWS_PER_TILE - nfull * zrows
        pltpu.async_copy(rb0.at[pl.ds(0, rem)],
                         accr.at[pl.ds(r0 + nfull * zrows, rem)], esem0)
        for r in range(nfull):
            pltpu.make_async_copy(rb0, accr.at[pl.ds(r0 + r * zrows, zrows)],
                                  esem0).wait()
        pltpu.make_async_copy(rb0.at[pl.ds(0, rem)],
                              accr.at[pl.ds(r0 + nfull * zrows, rem)], esem0).wait()

    def batch_gather(tab, layer):
        d0 = pltpu.async_copy(tab.at[ib640], rb0, gsem0)
        d1 = pltpu.async_copy(tab.at[ib128], rb1.at[pl.ds(0, 128)], gsem1)
        d0.wait()
        d1.wait()
        base = s * BPT

        @pl.when(c == 0)
        def _():
            pltpu.sync_copy(rb0, out_lo.at[layer, pl.ds(base, 640)])
            pltpu.sync_copy(rb1.at[pl.ds(0, 128)],
                            out_lo.at[layer, pl.ds(base + 640, 128)])

        @pl.when(c == 1)
        def _():
            pltpu.sync_copy(rb0, out_hi.at[layer, pl.ds(base, 640)])
            pltpu.sync_copy(rb1.at[pl.ds(0, 128)],
                            out_hi.at[layer, pl.ds(base + 640, 128)])

    def pipeline(tab, accr):
        _edge_pipeline(s, tab, accr, src_hbm, dst_hbm, w_hbm,
                       rb, sb, db, wb, esem, gsem, ssem)

    @pl.when(c == 0)
    def _():
        pltpu.async_copy(t_lo.at[pl.ds(r0, ROWS_PER_TILE)],
                         ta.at[pl.ds(r0, ROWS_PER_TILE)], ssem0)

    @pl.when(c == 1)
    def _():
        pltpu.async_copy(t_hi.at[pl.ds(r0, ROWS_PER_TILE)],
                         ta.at[pl.ds(r0, ROWS_PER_TILE)], ssem0)

    pltpu.sync_copy(idx_hbm.at[pl.ds(s * BPT, 640)], ib640)
    pltpu.sync_copy(idx_hbm.at[pl.ds(s * BPT + 640, 128)], ib128)
    zero_acc(tb)

    @pl.when(c == 0)
    def _():
        pltpu.make_async_copy(t_lo.at[pl.ds(r0, ROWS_PER_TILE)],
                              ta.at[pl.ds(r0, ROWS_PER_TILE)], ssem0).wait()

    @pl.when(c == 1)
    def _():
        pltpu.make_async_copy(t_hi.at[pl.ds(r0, ROWS_PER_TILE)],
                              ta.at[pl.ds(r0, ROWS_PER_TILE)], ssem0).wait()

    plsc.subcore_barrier()

    batch_gather(ta, 0)
    pipeline(ta, tb)
    batch_gather(tb, 1)
    zero_acc(ta)
    plsc.subcore_barrier()
    pipeline(tb, ta)
    batch_gather(ta, 2)
    zero_acc(tb)
    plsc.subcore_barrier()
    pipeline(ta, tb)
    batch_gather(tb, 3)


_mega = pl.kernel(
    _mega_body,
    out_type=(jax.ShapeDtypeStruct((4, B3, DH), jnp.float32),
              jax.ShapeDtypeStruct((4, B3, DH), jnp.float32)),
    mesh=_mesh,
    compiler_params=_sc_params,
    scratch_types=[
        pltpu.VMEM_SHARED((N_PAD, DH), jnp.float32),
        pltpu.VMEM_SHARED((N_PAD, DH), jnp.float32),
        pltpu.VMEM((CHUNK_ROWS * 128, DH), jnp.float32),
        pltpu.VMEM((CHUNK_ROWS * 128, DH), jnp.float32),
        pltpu.VMEM((CHUNK_ROWS * 128,), jnp.int32),
        pltpu.VMEM((CHUNK_ROWS * 128,), jnp.int32),
        pltpu.VMEM((CHUNK_ROWS * 128,), jnp.int32),
        pltpu.VMEM((CHUNK_ROWS * 128,), jnp.int32),
        pltpu.VMEM((CHUNK_ROWS * 128,), jnp.int32),
        pltpu.VMEM((CHUNK_ROWS * 128,), jnp.int32),
        pltpu.VMEM((CHUNK_ROWS * 128,), jnp.int32),
        pltpu.VMEM((CHUNK_ROWS * 128,), jnp.int32),
        pltpu.VMEM((CHUNK_ROWS * 128,), jnp.float32),
        pltpu.VMEM((CHUNK_ROWS * 128,), jnp.float32),
        pltpu.VMEM((CHUNK_ROWS * 128,), jnp.float32),
        pltpu.VMEM((CHUNK_ROWS * 128,), jnp.float32),
        pltpu.VMEM((640,), jnp.int32),
        pltpu.VMEM((128,), jnp.int32),
        pltpu.SemaphoreType.DMA,
        pltpu.SemaphoreType.DMA,
        pltpu.SemaphoreType.DMA,
        pltpu.SemaphoreType.DMA,
        pltpu.SemaphoreType.DMA,
        pltpu.SemaphoreType.DMA,
        pltpu.SemaphoreType.DMA,
        pltpu.SemaphoreType.DMA,
    ],
)


_BR = 1024


def _loss_body(ul, pl_, nl, uh, ph, nh, o_ref):
    i = pl.program_id(0)

    def avg(r):
        x = r[...]
        return (x[0] + x[1] + x[2] + x[3]) * 0.25

    ue = jnp.concatenate([avg(ul), avg(uh)], axis=1)
    pe = jnp.concatenate([avg(pl_), avg(ph)], axis=1)
    ne = jnp.concatenate([avg(nl), avg(nh)], axis=1)
    ps = jnp.sum(ue * pe, axis=1, keepdims=True)
    ns = jnp.sum(ue * ne, axis=1, keepdims=True)
    x = ns - ps
    sp = jnp.maximum(x, 0.0) + jnp.log1p(jnp.exp(-jnp.abs(x)))
    reg = (jnp.sum(ul[0] ** 2) + jnp.sum(uh[0] ** 2)
           + jnp.sum(pl_[0] ** 2) + jnp.sum(ph[0] ** 2)
           + jnp.sum(nl[0] ** 2) + jnp.sum(nh[0] ** 2))
    part = jnp.sum(sp) / float(BATCH) + (1e-4 * 0.5 / float(BATCH)) * reg

    @pl.when(i == 0)
    def _():
        o_ref[...] = jnp.zeros((1, 1), jnp.float32)

    o_ref[...] = o_ref[...] + jnp.full((1, 1), part, jnp.float32)


def _loss(blo, bhi):
    nb = BATCH // _BR
    bs = (4, _BR, DH)
    specs = []
    for off in (0, nb, 2 * nb):
        specs.append(pl.BlockSpec(bs, lambda i, off=off: (0, off + i, 0)))
    return pl.pallas_call(
        _loss_body,
        grid=(nb,),
        in_specs=[specs[0], specs[1], specs[2]] * 2,
        out_specs=pl.BlockSpec((1, 1), lambda i: (0, 0)),
        out_shape=jax.ShapeDtypeStruct((1, 1), jnp.float32),
    )(blo, blo, blo, bhi, bhi, bhi)


def kernel(user_emb, item_emb, edge_weight, edge_index, users, pos, neg):
    t0 = jnp.concatenate(
        [user_emb, item_emb,
         jnp.zeros((N_PAD - N_NODES, D), jnp.float32)], axis=0)
    t0l, t0h = t0[:, :DH], t0[:, DH:]
    pad = E_PAD - N_EDGES
    src = jnp.concatenate([edge_index[0], jnp.zeros((pad,), jnp.int32)])
    dst = jnp.concatenate([edge_index[1], jnp.zeros((pad,), jnp.int32)])
    w = jnp.concatenate([edge_weight, jnp.zeros((pad,), jnp.float32)])
    idx = jnp.concatenate([users, pos + N_USERS, neg + N_USERS])

    blo, bhi = _mega(t0l, t0h, src, dst, w, idx)
    out = _loss(blo, bhi)
    return out[0, 0]
